# baseline (speedup 1.0000x reference)
import functools

import jax
import jax.numpy as jnp
from jax import lax
from jax.experimental import pallas as pl
from jax.experimental.pallas import tpu as pltpu
from jax.experimental.pallas import tpu_sc as plsc

BATCH = 16384
EMB = 32
CHUNK = 128
NC, NS = 2, 16
NW = NC * NS
B_PER_W = BATCH // NW
N_CHUNKS = B_PER_W // CHUNK


def _sc_gather(user_table, task_table, uid, tid):
    mesh = plsc.VectorSubcoreMesh(core_axis_name="c", subcore_axis_name="s")

    @functools.partial(
        pl.kernel,
        mesh=mesh,
        compiler_params=pltpu.CompilerParams(use_tc_tiling_on_sc=False),
        out_type=[
            jax.ShapeDtypeStruct((BATCH, EMB), jnp.float32),
            jax.ShapeDtypeStruct((BATCH, EMB), jnp.float32),
        ],
        scratch_types=[
            pltpu.VMEM((B_PER_W,), jnp.int32),
            pltpu.VMEM((B_PER_W,), jnp.int32),
            pltpu.VMEM((B_PER_W, EMB), jnp.float32),
            pltpu.VMEM((B_PER_W, EMB), jnp.float32),
            pltpu.SemaphoreType.DMA,
            pltpu.SemaphoreType.DMA,
        ],
    )
    def gather_kernel(ut_hbm, tt_hbm, uid_hbm, tid_hbm, ue_out, te_out,
                      uidx_v, tidx_v, urows_v, trows_v, usem, tsem):
        wid = lax.axis_index("s") * NC + lax.axis_index("c")
        base = wid * B_PER_W
        pltpu.sync_copy(uid_hbm.at[pl.ds(base, B_PER_W)], uidx_v)
        pltpu.sync_copy(tid_hbm.at[pl.ds(base, B_PER_W)], tidx_v)
        ops = []
        for j in range(N_CHUNKS):
            ops.append(pltpu.async_copy(
                ut_hbm.at[uidx_v.at[pl.ds(j * CHUNK, CHUNK)]],
                urows_v.at[pl.ds(j * CHUNK, CHUNK)], usem))
            ops.append(pltpu.async_copy(
                tt_hbm.at[tidx_v.at[pl.ds(j * CHUNK, CHUNK)]],
                trows_v.at[pl.ds(j * CHUNK, CHUNK)], tsem))
        for op in ops:
            op.wait()
        pltpu.sync_copy(urows_v, ue_out.at[pl.ds(base, B_PER_W)])
        pltpu.sync_copy(trows_v, te_out.at[pl.ds(base, B_PER_W)])

    return gather_kernel(user_table, task_table, uid, tid)


_BLK = 2048


def _mlp_body(ue, te, w1u, w1t, b1, w2, b2, w3, b3, out_ref):
    h1 = jnp.dot(ue[...], w1u[...], preferred_element_type=jnp.float32)
    h1 += jnp.dot(te[...], w1t[...], preferred_element_type=jnp.float32)
    h1 = jnp.maximum(h1 + b1[...], 0.0)
    h2 = jnp.dot(h1, w2[...], preferred_element_type=jnp.float32)
    h2 = jnp.maximum(h2 + b2[...], 0.0)
    out_ref[...] = jnp.dot(h2, w3[...], preferred_element_type=jnp.float32) + b3[...]


def _mlp(ue, te, w1u, w1t, b1, w2, b2, w3, b3):
    grid = (BATCH // _BLK,)
    whole = lambda i: (0, 0)
    return pl.pallas_call(
        _mlp_body,
        grid=grid,
        in_specs=[
            pl.BlockSpec((_BLK, EMB), lambda i: (i, 0)),
            pl.BlockSpec((_BLK, EMB), lambda i: (i, 0)),
            pl.BlockSpec((EMB, 256), whole),
            pl.BlockSpec((EMB, 256), whole),
            pl.BlockSpec((1, 256), whole),
            pl.BlockSpec((256, 64), whole),
            pl.BlockSpec((1, 64), whole),
            pl.BlockSpec((64, 1), whole),
            pl.BlockSpec((1, 1), whole),
        ],
        out_specs=pl.BlockSpec((_BLK, 1), lambda i: (i, 0)),
        out_shape=jax.ShapeDtypeStruct((BATCH, 1), jnp.float32),
    )(ue, te, w1u, w1t, b1, w2, b2, w3, b3)


def kernel(user_id, task_id, user_table, task_table, W1, b1, W2, b2, W3, b3):
    uid = user_id.astype(jnp.int32)
    tid = task_id.astype(jnp.int32)
    ue, te = _sc_gather(user_table, task_table, uid, tid)
    return _mlp(ue, te, W1[:EMB], W1[EMB:], b1.reshape(1, 256),
                W2, b2.reshape(1, 64), W3, b3.reshape(1, 1))


# speedup vs baseline: 1.5454x; 1.5454x over previous
import functools

import jax
import jax.numpy as jnp
from jax import lax
from jax.experimental import pallas as pl
from jax.experimental.pallas import tpu as pltpu
from jax.experimental.pallas import tpu_sc as plsc

BATCH = 16384
EMB = 32
NC, NS = 2, 16
NW = NC * NS
B_PER_W = BATCH // NW
HALF = B_PER_W // 2
WAVE = 64
N_WAVES = HALF // WAVE


def _sc_gather(user_table, task_table, uid, tid):
    mesh = plsc.VectorSubcoreMesh(core_axis_name="c", subcore_axis_name="s")

    @functools.partial(
        pl.kernel,
        mesh=mesh,
        out_type=[
            jax.ShapeDtypeStruct((BATCH, EMB), jnp.float32),
            jax.ShapeDtypeStruct((BATCH, EMB), jnp.float32),
        ],
        scratch_types=[
            pltpu.VMEM((B_PER_W,), jnp.int32),
            pltpu.VMEM((B_PER_W,), jnp.int32),
            pltpu.VMEM((HALF, EMB), jnp.float32),
            pltpu.VMEM((HALF, EMB), jnp.float32),
            pltpu.SemaphoreType.DMA,
            pltpu.SemaphoreType.DMA,
        ],
    )
    def gather_kernel(ut_hbm, tt_hbm, uid_hbm, tid_hbm, ue_out, te_out,
                      uidx_v, tidx_v, urows_v, trows_v, usem, tsem):
        wid = lax.axis_index("s") * NC + lax.axis_index("c")
        base = wid * B_PER_W
        pltpu.sync_copy(uid_hbm.at[pl.ds(base, B_PER_W)], uidx_v)
        pltpu.sync_copy(tid_hbm.at[pl.ds(base, B_PER_W)], tidx_v)

        def half(h):
            hbase = h * HALF

            def wave(w, _):
                wbase = w * WAVE
                for g in range(WAVE // 16):
                    iu = uidx_v[pl.ds(hbase + wbase + g * 16, 16)]
                    it = tidx_v[pl.ds(hbase + wbase + g * 16, 16)]
                    for k in range(16):
                        pltpu.async_copy(
                            ut_hbm.at[pl.ds(iu[k], 1)],
                            urows_v.at[pl.ds(wbase + g * 16 + k, 1)], usem)
                        pltpu.async_copy(
                            tt_hbm.at[pl.ds(it[k], 1)],
                            trows_v.at[pl.ds(wbase + g * 16 + k, 1)], tsem)
                pltpu.make_async_copy(
                    ut_hbm.at[pl.ds(0, WAVE)],
                    urows_v.at[pl.ds(w * WAVE, WAVE)], usem).wait()
                pltpu.make_async_copy(
                    tt_hbm.at[pl.ds(0, WAVE)],
                    trows_v.at[pl.ds(w * WAVE, WAVE)], tsem).wait()
                return ()

            lax.fori_loop(0, N_WAVES, wave, (), unroll=False)
            pltpu.sync_copy(urows_v, ue_out.at[pl.ds(base + hbase, HALF)])
            pltpu.sync_copy(trows_v, te_out.at[pl.ds(base + hbase, HALF)])

        half(0)
        half(1)

    return gather_kernel(user_table, task_table, uid, tid)


_BLK = 2048


def _mlp_body(ue, te, w1u, w1t, b1, w2, b2, w3, b3, out_ref):
    h1 = jnp.dot(ue[...], w1u[...], preferred_element_type=jnp.float32)
    h1 += jnp.dot(te[...], w1t[...], preferred_element_type=jnp.float32)
    h1 = jnp.maximum(h1 + b1[...], 0.0)
    h2 = jnp.dot(h1, w2[...], preferred_element_type=jnp.float32)
    h2 = jnp.maximum(h2 + b2[...], 0.0)
    out_ref[...] = jnp.dot(h2, w3[...], preferred_element_type=jnp.float32) + b3[...]


def _mlp(ue, te, w1u, w1t, b1, w2, b2, w3, b3):
    grid = (BATCH // _BLK,)
    whole = lambda i: (0, 0)
    return pl.pallas_call(
        _mlp_body,
        grid=grid,
        in_specs=[
            pl.BlockSpec((_BLK, EMB), lambda i: (i, 0)),
            pl.BlockSpec((_BLK, EMB), lambda i: (i, 0)),
            pl.BlockSpec((EMB, 256), whole),
            pl.BlockSpec((EMB, 256), whole),
            pl.BlockSpec((1, 256), whole),
            pl.BlockSpec((256, 64), whole),
            pl.BlockSpec((1, 64), whole),
            pl.BlockSpec((64, 1), whole),
            pl.BlockSpec((1, 1), whole),
        ],
        out_specs=pl.BlockSpec((_BLK, 1), lambda i: (i, 0)),
        out_shape=jax.ShapeDtypeStruct((BATCH, 1), jnp.float32),
    )(ue, te, w1u, w1t, b1, w2, b2, w3, b3)


def kernel(user_id, task_id, user_table, task_table, W1, b1, W2, b2, W3, b3):
    uid = user_id.astype(jnp.int32)
    tid = task_id.astype(jnp.int32)
    ue, te = _sc_gather(user_table, task_table, uid, tid)
    return _mlp(ue, te, W1[:EMB], W1[EMB:], b1.reshape(1, 256),
                W2, b2.reshape(1, 64), W3, b3.reshape(1, 1))
